# Initial kernel scaffold; baseline (speedup 1.0000x reference)
import jax
import jax.numpy as jnp
from jax.experimental import pallas as pl


def kernel(x, weight):
    raise NotImplementedError("write your pallas kernel here")


# speedup vs baseline: 2.4888x; 2.4888x over previous
import functools

import jax
import jax.numpy as jnp
from jax import lax
from jax.experimental import pallas as pl
from jax.experimental.pallas import tpu as pltpu
from jax.experimental.pallas import tpu_sc as plsc

_VOCAB = 1000000
_EMBED_DIM = 16
_BATCH = 16384
_HIST = 200

_B = _BATCH * _HIST
_IDXW = 128
_K = 16
_CHUNK = _K * _IDXW

_NC = 2
_NS = 16
_NW = _NC * _NS
_PER_W = _B // _NW
_GROUPS = _PER_W // _CHUNK

assert _PER_W % _CHUNK == 0


def _body(w_hbm, idx_hbm, out_hbm, idx_v, rows_v, sem_g):
    wid = lax.axis_index("s") * _NC + lax.axis_index("c")
    row0 = wid * _PER_W
    irow0 = row0 // _IDXW

    @pl.loop(0, _GROUPS)
    def _group(g):
        pltpu.sync_copy(
            idx_hbm.at[pl.ds(pl.multiple_of(irow0 + g * _K, 8), _K)], idx_v
        )
        copies = []
        for j in range(_K):
            copies.append(
                pltpu.async_copy(
                    w_hbm.at[idx_v.at[j]],
                    rows_v.at[pl.ds(j * _IDXW, _IDXW)],
                    sem_g,
                )
            )
        for c in copies:
            c.wait()
        pltpu.sync_copy(
            rows_v,
            out_hbm.at[pl.ds(pl.multiple_of(row0 + g * _CHUNK, 8), _CHUNK)],
        )


_embed = pl.kernel(
    _body,
    out_type=jax.ShapeDtypeStruct((_B, _EMBED_DIM), jnp.float32),
    mesh=plsc.VectorSubcoreMesh(core_axis_name="c", subcore_axis_name="s"),
    scratch_types=[
        pltpu.VMEM((_K, _IDXW), jnp.int32),
        pltpu.VMEM((_CHUNK, _EMBED_DIM), jnp.float32),
        pltpu.SemaphoreType.DMA,
    ],
    compiler_params=pltpu.CompilerParams(use_tc_tiling_on_sc=False),
)


@jax.jit
def kernel(x, weight):
    idx = x.reshape(_B // _IDXW, _IDXW).astype(jnp.int32)
    out = _embed(weight, idx)
    return out.reshape(_BATCH, _HIST, _EMBED_DIM)


# speedup vs baseline: 2.5319x; 1.0173x over previous
import jax
import jax.numpy as jnp
from jax import lax
from jax.experimental import pallas as pl
from jax.experimental.pallas import tpu as pltpu
from jax.experimental.pallas import tpu_sc as plsc

_VOCAB = 1000000
_EMBED_DIM = 16
_BATCH = 16384
_HIST = 200

_B = _BATCH * _HIST
_IDXW = 128
_K = 16
_CHUNK = _K * _IDXW

_NC = 2
_NS = 16
_NW = _NC * _NS
_PER_W = _B // _NW
_GROUPS = _PER_W // _CHUNK

assert _PER_W % _CHUNK == 0


def _body(w_hbm, idx_hbm, out_hbm, idx_v, rows_v, sem_i, sem_g, sem_o):
    wid = lax.axis_index("s") * _NC + lax.axis_index("c")
    row0 = wid * _PER_W
    irow0 = row0 // _IDXW

    def idx_src(g):
        return idx_hbm.at[pl.ds(pl.multiple_of(irow0 + g * _K, 8), _K)]

    pltpu.async_copy(idx_src(0), idx_v.at[0], sem_i)

    @pl.loop(0, _GROUPS)
    def _group(g):
        p = lax.rem(g, 2)
        q = 1 - p
        pltpu.make_async_copy(idx_src(g), idx_v.at[p], sem_i).wait()
        copies = []
        for j in range(_K):
            copies.append(
                pltpu.async_copy(
                    w_hbm.at[idx_v.at[p].at[j]],
                    rows_v.at[p].at[pl.ds(j * _IDXW, _IDXW)],
                    sem_g,
                )
            )
        @pl.when(g + 1 < _GROUPS)
        def _():
            pltpu.async_copy(idx_src(g + 1), idx_v.at[q], sem_i)

        for c in copies:
            c.wait()
        @pl.when(g >= 1)
        def _():
            pltpu.make_async_copy(
                rows_v.at[q], out_hbm.at[pl.ds(0, _CHUNK)], sem_o
            ).wait()

        pltpu.async_copy(
            rows_v.at[p],
            out_hbm.at[pl.ds(pl.multiple_of(row0 + g * _CHUNK, 8), _CHUNK)],
            sem_o,
        )

    pltpu.make_async_copy(
        rows_v.at[0], out_hbm.at[pl.ds(0, _CHUNK)], sem_o
    ).wait()


_embed = pl.kernel(
    _body,
    out_type=jax.ShapeDtypeStruct((_B, _EMBED_DIM), jnp.float32),
    mesh=plsc.VectorSubcoreMesh(core_axis_name="c", subcore_axis_name="s"),
    scratch_types=[
        pltpu.VMEM((2, _K, _IDXW), jnp.int32),
        pltpu.VMEM((2, _CHUNK, _EMBED_DIM), jnp.float32),
        pltpu.SemaphoreType.DMA,
        pltpu.SemaphoreType.DMA,
        pltpu.SemaphoreType.DMA,
    ],
    compiler_params=pltpu.CompilerParams(use_tc_tiling_on_sc=False),
)


@jax.jit
def kernel(x, weight):
    idx = x.reshape(_B // _IDXW, _IDXW).astype(jnp.int32)
    out = _embed(weight, idx)
    return out.reshape(_BATCH, _HIST, _EMBED_DIM)
